# baseline (speedup 1.0000x reference)
import functools
import math

import jax
import jax.numpy as jnp
from jax import lax
from jax.experimental import pallas as pl
from jax.experimental.pallas import tpu as pltpu
from jax.experimental.pallas import tpu_sc as plsc

D_MODEL = 64
SCALE = math.sqrt(D_MODEL)
NC, NS, L = 2, 16, 16
NW = NC * NS
CHUNK = 128


def _make_sc_lookup(B, V, D):
    assert B % (NW * CHUNK) == 0
    b_per_w = B // NW
    nchunks = b_per_w // CHUNK
    assert nchunks % 2 == 0
    mesh = plsc.VectorSubcoreMesh(core_axis_name="c", subcore_axis_name="s")

    @functools.partial(
        pl.kernel,
        mesh=mesh,
        out_type=jax.ShapeDtypeStruct((B, D), jnp.float32),
        scratch_types=[
            pltpu.VMEM((nchunks, CHUNK), jnp.int32),
            pltpu.VMEM((CHUNK, D), jnp.float32),
            pltpu.VMEM((CHUNK, D), jnp.float32),
            pltpu.SemaphoreType.DMA,
            pltpu.SemaphoreType.DMA,
        ],
        compiler_params=pltpu.CompilerParams(use_tc_tiling_on_sc=False),
    )
    def lookup(x_hbm, table_hbm, out_hbm, idx_v, buf0, buf1, sem0, sem1):
        wid = lax.axis_index("s") * NC + lax.axis_index("c")
        pltpu.sync_copy(x_hbm.at[pl.ds(wid * nchunks, nchunks)], idx_v)

        bufs = (buf0, buf1)
        sems = (sem0, sem1)
        out_base = wid * b_per_w

        def start_gather(g, b):
            pltpu.async_copy(table_hbm.at[idx_v.at[g]], bufs[b], sems[b])

        def wait_gather(g, b):
            pltpu.make_async_copy(
                table_hbm.at[idx_v.at[g]], bufs[b], sems[b]
            ).wait()

        def scale_rows(buf):
            def row(i, c):
                for j in range(D // L):
                    sl = pl.ds(j * L, L)
                    buf[i, sl] = buf[i, sl] * SCALE
                return c

            lax.fori_loop(0, CHUNK, row, 0)

        def process(g, b):
            wait_gather(g, b)
            scale_rows(bufs[b])
            pltpu.sync_copy(
                bufs[b], out_hbm.at[pl.ds(out_base + g * CHUNK, CHUNK)]
            )

        start_gather(0, 0)

        def pair(p, c):
            g0 = 2 * p
            start_gather(g0 + 1, 1)
            process(g0, 0)

            @pl.when(g0 + 2 < nchunks)
            def _():
                start_gather(g0 + 2, 0)

            process(g0 + 1, 1)
            return c

        lax.fori_loop(0, nchunks // 2, pair, 0)

    return lookup


def kernel(x, table):
    S, T = x.shape
    V, D = table.shape
    B = S * T
    x_flat = x.reshape(B // CHUNK, CHUNK)
    out = _make_sc_lookup(B, V, D)(x_flat, table)
    return out.reshape(S, T, D)


# speedup vs baseline: 1.0585x; 1.0585x over previous
import functools
import math

import jax
import jax.numpy as jnp
from jax import lax
from jax.experimental import pallas as pl
from jax.experimental.pallas import tpu as pltpu
from jax.experimental.pallas import tpu_sc as plsc

D_MODEL = 64
SCALE = math.sqrt(D_MODEL)
NC, NS, L = 2, 16, 16
NW = NC * NS
QROWS = 128
QPC = 4
CHUNK = QROWS * QPC
UNROLL = 4


def _make_sc_lookup(B, V, D):
    assert B % (NW * CHUNK) == 0
    b_per_w = B // NW
    nchunks = b_per_w // CHUNK
    nq = b_per_w // QROWS
    assert nchunks % 2 == 0
    mesh = plsc.VectorSubcoreMesh(core_axis_name="c", subcore_axis_name="s")

    @functools.partial(
        pl.kernel,
        mesh=mesh,
        out_type=jax.ShapeDtypeStruct((B, D), jnp.float32),
        scratch_types=[
            pltpu.VMEM((nq, QROWS), jnp.int32),
            pltpu.VMEM((CHUNK, D), jnp.float32),
            pltpu.VMEM((CHUNK, D), jnp.float32),
            pltpu.SemaphoreType.DMA,
            pltpu.SemaphoreType.DMA,
        ],
        compiler_params=pltpu.CompilerParams(use_tc_tiling_on_sc=False),
    )
    def lookup(x_hbm, table_hbm, out_hbm, idx_v, buf0, buf1, sem0, sem1):
        wid = lax.axis_index("s") * NC + lax.axis_index("c")
        pltpu.sync_copy(x_hbm.at[pl.ds(wid * nq, nq)], idx_v)

        bufs = (buf0, buf1)
        sems = (sem0, sem1)
        out_base = wid * b_per_w

        def start_gather(g, b):
            for q in range(QPC):
                pltpu.async_copy(
                    table_hbm.at[idx_v.at[g * QPC + q]],
                    bufs[b].at[pl.ds(q * QROWS, QROWS)],
                    sems[b],
                )

        def wait_gather(g, b):
            for q in range(QPC):
                pltpu.make_async_copy(
                    table_hbm.at[idx_v.at[g * QPC + q]],
                    bufs[b].at[pl.ds(q * QROWS, QROWS)],
                    sems[b],
                ).wait()

        def scale_rows(buf):
            def rows(i, c):
                for r in range(UNROLL):
                    for j in range(D // L):
                        ix = i * UNROLL + r
                        sl = pl.ds(j * L, L)
                        buf[ix, sl] = buf[ix, sl] * SCALE
                return c

            lax.fori_loop(0, CHUNK // UNROLL, rows, 0)

        def process(g, b):
            wait_gather(g, b)
            scale_rows(bufs[b])
            pltpu.sync_copy(
                bufs[b], out_hbm.at[pl.ds(out_base + g * CHUNK, CHUNK)]
            )

        start_gather(0, 0)

        def pair(p, c):
            g0 = 2 * p
            start_gather(g0 + 1, 1)
            process(g0, 0)

            @pl.when(g0 + 2 < nchunks)
            def _():
                start_gather(g0 + 2, 0)

            process(g0 + 1, 1)
            return c

        lax.fori_loop(0, nchunks // 2, pair, 0)

    return lookup


def kernel(x, table):
    S, T = x.shape
    V, D = table.shape
    B = S * T
    x_flat = x.reshape(B // QROWS, QROWS)
    out = _make_sc_lookup(B, V, D)(x_flat, table)
    return out.reshape(S, T, D)
